# baseline (speedup 1.0000x reference)
import functools

import jax
import jax.numpy as jnp
import numpy as np
from jax import lax
from jax.experimental import pallas as pl
from jax.experimental.pallas import tpu as pltpu
from jax.experimental.pallas import tpu_sc as plsc

_K = 256
_N = 2048
_H = 12
_DH = 64
_EPS = 1e-06
_MASK_VAL = float(np.finfo(np.float32).max) / 2
_NEG_BIG = -1e38

_NC = 2
_NS = 16
_NW = _NC * _NS
_CHUNK = 8
_CPH = 33
_NCH = _H * _CPH
_CPW = 13
_IDXN = _NW * _CPW * _CHUNK


def _sample_body(cls_ref, val_ref, maskf_ref, g2_ref, uniq_ref, maskout_ref,
                 idx_ref):
    v = val_ref[...]
    norms = jnp.sqrt(jnp.sum(v * v, axis=-1))
    s = jnp.sum(cls_ref[...] * norms, axis=0, keepdims=True)

    it = lax.broadcasted_iota(jnp.int32, (1, _N), 1)
    validf = (it >= 1).astype(jnp.float32)
    total = jnp.sum(s * validf)
    logits = jnp.log(s / (total + _EPS) + _EPS)
    logits = jnp.where(maskf_ref[...] > 0.5, logits, -_MASK_VAL)

    score = logits + g2_ref[...]
    m = jnp.max(score, axis=1, keepdims=True)
    ti = lax.broadcasted_iota(jnp.int32, (_K, _N), 1)
    sampled = jnp.min(jnp.where(score == m, ti, _N * 2), axis=1,
                      keepdims=True)

    presentf = jnp.max((sampled == ti).astype(jnp.float32), axis=0,
                       keepdims=True)
    n_uniq = jnp.sum(presentf)

    it0 = lax.broadcasted_iota(jnp.int32, (_N, _N), 0)
    it1 = lax.broadcasted_iota(jnp.int32, (_N, _N), 1)
    tri = (it1 <= it0).astype(jnp.float32)
    ranks_col = lax.dot_general(tri, presentf, (((1,), (1,)), ((), ())),
                                preferred_element_type=jnp.float32)
    ranks_excl = jnp.concatenate(
        [jnp.zeros((1, 1), jnp.float32), ranks_col[: _N - 1, :]], axis=0)
    present_col = ranks_col - ranks_excl

    slot_col = ranks_col + (float(_K) - 1.0 - n_uniq)
    jj = lax.broadcasted_iota(jnp.int32, (_N, _K), 1).astype(jnp.float32)
    tvals = lax.broadcasted_iota(jnp.int32, (_N, _K), 0).astype(jnp.float32)
    hit = (slot_col == jj) & (present_col > 0.5)
    uniq_row = jnp.sum(jnp.where(hit, tvals, 0.0), axis=0,
                       keepdims=True)
    uniq_i = uniq_row.astype(jnp.int32)
    uniq257 = jnp.concatenate(
        [jnp.zeros((1, 1), jnp.int32), uniq_i], axis=1)
    mask257 = jnp.concatenate(
        [jnp.ones((1, 1), jnp.int32), (uniq_i != 0).astype(jnp.int32)], axis=1)

    uniq_ref[...] = uniq257
    maskout_ref[...] = mask257
    tail = uniq257[:, _K:_K + 1]
    uniq_ext = jnp.concatenate(
        [uniq257, jnp.broadcast_to(tail, (1, _CPH * _CHUNK - _K - 1))],
        axis=1)
    hh = lax.broadcasted_iota(jnp.int32, (_H, _CPH * _CHUNK), 0)
    idx_ref[...] = hh * _N + uniq_ext


def _sampling_call(cls_row, value3, maskf, g2):
    return pl.pallas_call(
        _sample_body,
        out_shape=[
            jax.ShapeDtypeStruct((1, _K + 1), jnp.int32),
            jax.ShapeDtypeStruct((1, _K + 1), jnp.int32),
            jax.ShapeDtypeStruct((_H, _CPH * _CHUNK), jnp.int32),
        ],
    )(cls_row, value3, maskf, g2)


@functools.cache
def _g2_const():
    with jax.ensure_compile_time_eval():
        gkey = jax.random.fold_in(jax.random.key(0), 123)
        u = jax.random.uniform(gkey, (1, _K, _N - 1), dtype=jnp.float32)
        gum = -jnp.log(-jnp.log(u + _EPS) + _EPS)
        g2 = jnp.concatenate(
            [jnp.full((_K, 1), _NEG_BIG, jnp.float32),
             gum.reshape(_K, _N - 1)], axis=1)
        return np.asarray(jax.device_get(g2))


@functools.cache
def _jidx_const():
    rows = np.minimum(np.arange(_CPH * _CHUNK), _K).astype(np.int32)
    pat = rows.reshape(_CPH, _CHUNK)
    c = np.arange(_NW * _CPW)
    g = pat[c % _CPH]
    return g.astype(np.int32).reshape(_NW, _CPW, _CHUNK)


@functools.cache
def _make_gather():
    @functools.partial(
        pl.kernel,
        mesh=plsc.VectorSubcoreMesh(core_axis_name="c", subcore_axis_name="s"),
        out_type=jax.ShapeDtypeStruct((1, _H, _K + 1, _N), jnp.float32),
        compiler_params=pltpu.CompilerParams(use_tc_tiling_on_sc=True),
        scratch_types=[
            pltpu.VMEM((_CPW * _CHUNK,), jnp.int32),
            pltpu.VMEM((_CPW, _CHUNK), jnp.int32),
            pltpu.VMEM((_CHUNK, _N), jnp.float32),
            pltpu.VMEM((_CHUNK, _N), jnp.float32),
            pltpu.SemaphoreType.DMA,
            pltpu.SemaphoreType.DMA,
            pltpu.SemaphoreType.DMA,
            pltpu.SemaphoreType.DMA,
        ],
    )
    def _gather_rows(table_hbm, idx_hbm, jidx_hbm, out_hbm, idx_v, jidx_v,
                     buf0, buf1, gsem0, gsem1, ssem0, ssem1):
        wid = lax.axis_index("s") * _NC + lax.axis_index("c")
        base = wid * _CPW * _CHUNK
        pltpu.sync_copy(idx_hbm.at[pl.ds(base, _CPW * _CHUNK)], idx_v)
        pltpu.sync_copy(jidx_hbm.at[wid], jidx_v)
        bufs = (buf0, buf1)
        gsems = (gsem0, gsem1)
        ssems = (ssem0, ssem1)

        def chunk_of(t):
            return wid * _CPW + t

        def gather_desc(t):
            return pltpu.make_async_copy(
                table_hbm.at[idx_v.at[pl.ds(t * _CHUNK, _CHUNK)]],
                bufs[t % 2], gsems[t % 2])

        def scatter_desc(t):
            h = chunk_of(t) // _CPH
            return pltpu.make_async_copy(
                bufs[t % 2], out_hbm.at[0, h].at[jidx_v.at[t]], ssems[t % 2])

        def start_gather(t):
            @pl.when(chunk_of(t) < _NCH)
            def _():
                gather_desc(t).start()

        def wait_gather(t):
            @pl.when(chunk_of(t) < _NCH)
            def _():
                gather_desc(t).wait()

        def start_scatter(t):
            @pl.when(chunk_of(t) < _NCH)
            def _():
                scatter_desc(t).start()

        def wait_scatter(t):
            @pl.when(chunk_of(t) < _NCH)
            def _():
                scatter_desc(t).wait()

        start_gather(0)
        for t in range(_CPW):
            if t >= 1:
                wait_scatter(t - 1)
            if t + 1 < _CPW:
                start_gather(t + 1)
            wait_gather(t)
            start_scatter(t)
        wait_scatter(_CPW - 1)

    return _gather_rows


def kernel(attn, value, mask):
    b, h, n, _ = attn.shape
    k = _K
    g2 = jnp.asarray(_g2_const())

    cls_row = attn[0, :, 0, :]
    value3 = value[0]
    maskf = mask.astype(jnp.float32)

    uniq257, mask257, idx264 = _sampling_call(cls_row, value3, maskf, g2)

    idx_flat = jnp.pad(idx264.reshape(_H * _CPH * _CHUNK),
                       (0, _IDXN - _H * _CPH * _CHUNK))
    jidx = jnp.asarray(_jidx_const())
    table = attn.reshape(h * n, n)

    new_attn = _make_gather()(table, idx_flat, jidx)
    return new_attn, mask257.astype(bool), uniq257


# speedup vs baseline: 1.1130x; 1.1130x over previous
import functools

import jax
import jax.numpy as jnp
import numpy as np
from jax import lax
from jax.experimental import pallas as pl
from jax.experimental.pallas import tpu as pltpu
from jax.experimental.pallas import tpu_sc as plsc

_K = 256
_N = 2048
_H = 12
_DH = 64
_EPS = 1e-06
_MASK_VAL = float(np.finfo(np.float32).max) / 2
_NEG_BIG = -1e38

_NC = 2
_NS = 16
_NW = _NC * _NS
_CHUNK = 8
_CPH = 33
_NCH = _H * _CPH
_CPW = 13
_IDXN = _NW * _CPW * _CHUNK


def _sample_body(cls_ref, val_ref, maskf_ref, g2_ref, uniq_ref, maskout_ref,
                 idx_ref):
    v = val_ref[...]
    norms = jnp.sqrt(jnp.sum(v * v, axis=-1))
    s = jnp.sum(cls_ref[...] * norms, axis=0, keepdims=True)

    it = lax.broadcasted_iota(jnp.int32, (1, _N), 1)
    validf = (it >= 1).astype(jnp.float32)
    total = jnp.sum(s * validf)
    logits = jnp.log(s / (total + _EPS) + _EPS)
    logits = jnp.where(maskf_ref[...] > 0.5, logits, -_MASK_VAL)

    score = logits + g2_ref[...]
    m = jnp.max(score, axis=1, keepdims=True)
    ti = lax.broadcasted_iota(jnp.int32, (_K, _N), 1)
    sampled = jnp.min(jnp.where(score == m, ti, _N * 2), axis=1,
                      keepdims=True)

    presentf = jnp.max((sampled == ti).astype(jnp.float32), axis=0,
                       keepdims=True)
    n_uniq = jnp.sum(presentf)

    it0 = lax.broadcasted_iota(jnp.int32, (_N, _N), 0)
    it1 = lax.broadcasted_iota(jnp.int32, (_N, _N), 1)
    tri = (it1 <= it0).astype(jnp.float32)
    ranks_col = lax.dot_general(tri, presentf, (((1,), (1,)), ((), ())),
                                preferred_element_type=jnp.float32)
    ranks_excl = jnp.concatenate(
        [jnp.zeros((1, 1), jnp.float32), ranks_col[: _N - 1, :]], axis=0)
    present_col = ranks_col - ranks_excl

    slot_col = ranks_col + (float(_K) - 1.0 - n_uniq)
    jj = lax.broadcasted_iota(jnp.int32, (_N, _K), 1).astype(jnp.float32)
    tvals = lax.broadcasted_iota(jnp.int32, (_N, _K), 0).astype(jnp.float32)
    hit = (slot_col == jj) & (present_col > 0.5)
    uniq_row = jnp.sum(jnp.where(hit, tvals, 0.0), axis=0,
                       keepdims=True)
    uniq_i = uniq_row.astype(jnp.int32)
    uniq257 = jnp.concatenate(
        [jnp.zeros((1, 1), jnp.int32), uniq_i], axis=1)
    mask257 = jnp.concatenate(
        [jnp.ones((1, 1), jnp.int32), (uniq_i != 0).astype(jnp.int32)], axis=1)

    uniq_ref[...] = uniq257
    maskout_ref[...] = mask257
    tail = uniq257[:, _K:_K + 1]
    uniq_ext = jnp.concatenate(
        [uniq257, jnp.broadcast_to(tail, (1, _CPH * _CHUNK - _K - 1))],
        axis=1)
    hh = lax.broadcasted_iota(jnp.int32, (_H, _CPH * _CHUNK), 0)
    idx_ref[...] = hh * _N + uniq_ext


def _sampling_call(cls_row, value3, maskf, g2):
    return pl.pallas_call(
        _sample_body,
        out_shape=[
            jax.ShapeDtypeStruct((1, _K + 1), jnp.int32),
            jax.ShapeDtypeStruct((1, _K + 1), jnp.int32),
            jax.ShapeDtypeStruct((_H, _CPH * _CHUNK), jnp.int32),
        ],
    )(cls_row, value3, maskf, g2)


@functools.cache
def _g2_const():
    with jax.ensure_compile_time_eval():
        gkey = jax.random.fold_in(jax.random.key(0), 123)
        u = jax.random.uniform(gkey, (1, _K, _N - 1), dtype=jnp.float32)
        gum = -jnp.log(-jnp.log(u + _EPS) + _EPS)
        g2 = jnp.concatenate(
            [jnp.full((_K, 1), _NEG_BIG, jnp.float32),
             gum.reshape(_K, _N - 1)], axis=1)
        return np.asarray(jax.device_get(g2))


@functools.cache
def _jidx_const():
    rows = np.minimum(np.arange(_CPH * _CHUNK), _K).astype(np.int32)
    pat = rows.reshape(_CPH, _CHUNK)
    c = np.arange(_NW * _CPW)
    g = pat[c % _CPH] + (c // _CPH).clip(max=_H - 1)[:, None] * (_CPH * _CHUNK)
    return g.astype(np.int32).reshape(_NW, _CPW, _CHUNK)


@functools.cache
def _make_gather():
    @functools.partial(
        pl.kernel,
        mesh=plsc.VectorSubcoreMesh(core_axis_name="c", subcore_axis_name="s"),
        out_type=jax.ShapeDtypeStruct((_H * _CPH * _CHUNK, _N), jnp.float32),
        scratch_types=[
            pltpu.VMEM((_CPW * _CHUNK,), jnp.int32),
            pltpu.VMEM((_CPW, _CHUNK), jnp.int32),
            pltpu.VMEM((_CHUNK, _N), jnp.float32),
            pltpu.VMEM((_CHUNK, _N), jnp.float32),
            pltpu.VMEM((_CHUNK, _N), jnp.float32),
            pltpu.VMEM((_CHUNK, _N), jnp.float32),
            pltpu.SemaphoreType.DMA,
            pltpu.SemaphoreType.DMA,
            pltpu.SemaphoreType.DMA,
            pltpu.SemaphoreType.DMA,
            pltpu.SemaphoreType.DMA,
            pltpu.SemaphoreType.DMA,
            pltpu.SemaphoreType.DMA,
            pltpu.SemaphoreType.DMA,
        ],
    )
    def _gather_rows(table_hbm, idx_hbm, jidx_hbm, out_hbm, idx_v, jidx_v,
                     buf0, buf1, buf2, buf3, gsem0, gsem1, gsem2, gsem3,
                     ssem0, ssem1, ssem2, ssem3):
        wid = lax.axis_index("s") * _NC + lax.axis_index("c")
        base = wid * _CPW * _CHUNK
        pltpu.sync_copy(idx_hbm.at[pl.ds(base, _CPW * _CHUNK)], idx_v)
        pltpu.sync_copy(jidx_hbm.at[wid], jidx_v)
        bufs = (buf0, buf1, buf2, buf3)
        gsems = (gsem0, gsem1, gsem2, gsem3)
        ssems = (ssem0, ssem1, ssem2, ssem3)

        def chunk_of(t):
            return wid * _CPW + t

        def gather_desc(t):
            return pltpu.make_async_copy(
                table_hbm.at[idx_v.at[pl.ds(t * _CHUNK, _CHUNK)]],
                bufs[t % 4], gsems[t % 4])

        def scatter_desc(t):
            return pltpu.make_async_copy(
                bufs[t % 4], out_hbm.at[jidx_v.at[t]], ssems[t % 4])

        def start_gather(t):
            @pl.when(chunk_of(t) < _NCH)
            def _():
                gather_desc(t).start()

        def wait_gather(t):
            @pl.when(chunk_of(t) < _NCH)
            def _():
                gather_desc(t).wait()

        def start_scatter(t):
            @pl.when(chunk_of(t) < _NCH)
            def _():
                scatter_desc(t).start()

        def wait_scatter(t):
            @pl.when(chunk_of(t) < _NCH)
            def _():
                scatter_desc(t).wait()

        start_gather(0)
        start_gather(1)
        for t in range(_CPW):
            if t >= 2:
                wait_scatter(t - 2)
            if t + 2 < _CPW:
                start_gather(t + 2)
            wait_gather(t)
            start_scatter(t)
        wait_scatter(_CPW - 2)
        wait_scatter(_CPW - 1)

    return _gather_rows


def kernel(attn, value, mask):
    b, h, n, _ = attn.shape
    k = _K
    g2 = jnp.asarray(_g2_const())

    cls_row = attn[0, :, 0, :]
    value3 = value[0]
    maskf = mask.astype(jnp.float32)

    uniq257, mask257, idx264 = _sampling_call(cls_row, value3, maskf, g2)

    idx_flat = jnp.pad(idx264.reshape(_H * _CPH * _CHUNK),
                       (0, _IDXN - _H * _CPH * _CHUNK))
    jidx = jnp.asarray(_jidx_const())
    table = attn.reshape(h * n, n)

    out = _make_gather()(table, idx_flat, jidx)
    new_attn = out.reshape(_H, _CPH * _CHUNK, n)[:, :k + 1, :].reshape(
        1, h, k + 1, n)
    return new_attn, mask257.astype(bool), uniq257


# speedup vs baseline: 1.1384x; 1.0228x over previous
import functools

import jax
import jax.numpy as jnp
import numpy as np
from jax import lax
from jax.experimental import pallas as pl
from jax.experimental.pallas import tpu as pltpu
from jax.experimental.pallas import tpu_sc as plsc

_K = 256
_N = 2048
_H = 12
_DH = 64
_EPS = 1e-06
_MASK_VAL = float(np.finfo(np.float32).max) / 2
_NEG_BIG = -1e38

_NC = 2
_NS = 16
_NW = _NC * _NS
_CHUNK = 8
_CPH = 33
_NCH = _H * _CPH
_CPW = 13
_IDXN = _NW * _CPW * _CHUNK


def _sample_body(cls_ref, val_ref, maskf_ref, g2_ref, uniq_ref, maskout_ref,
                 idx_ref):
    cls = cls_ref[...]
    s = jnp.zeros((1, _N), jnp.float32)
    for hh0 in range(_H):
        vt = jnp.swapaxes(val_ref[hh0], 0, 1)
        n2 = jnp.sum(vt * vt, axis=0, keepdims=True)
        s = s + cls[hh0:hh0 + 1, :] * jnp.sqrt(n2)

    it = lax.broadcasted_iota(jnp.int32, (1, _N), 1)
    validf = (it >= 1).astype(jnp.float32)
    total = jnp.sum(s * validf)
    logits = jnp.log(s / (total + _EPS) + _EPS)
    logits = jnp.where(maskf_ref[...] > 0.5, logits, -_MASK_VAL)

    score = logits + g2_ref[...]
    m = jnp.max(score, axis=1, keepdims=True)
    ti = lax.broadcasted_iota(jnp.int32, (_K, _N), 1)
    sampled = jnp.min(jnp.where(score == m, ti, _N * 2), axis=1,
                      keepdims=True)

    presentf = jnp.max((sampled == ti).astype(jnp.float32), axis=0,
                       keepdims=True)
    n_uniq = jnp.sum(presentf)

    it0 = lax.broadcasted_iota(jnp.int32, (_N, _N), 0)
    it1 = lax.broadcasted_iota(jnp.int32, (_N, _N), 1)
    tri = (it1 <= it0).astype(jnp.float32)
    ranks_col = lax.dot_general(tri, presentf, (((1,), (1,)), ((), ())),
                                preferred_element_type=jnp.float32)
    ranks_excl = jnp.concatenate(
        [jnp.zeros((1, 1), jnp.float32), ranks_col[: _N - 1, :]], axis=0)
    present_col = ranks_col - ranks_excl

    slot_col = ranks_col + (float(_K) - 1.0 - n_uniq)
    jj = lax.broadcasted_iota(jnp.int32, (_N, _K), 1).astype(jnp.float32)
    tvals = lax.broadcasted_iota(jnp.int32, (_N, _K), 0).astype(jnp.float32)
    hit = (slot_col == jj) & (present_col > 0.5)
    uniq_row = jnp.sum(jnp.where(hit, tvals, 0.0), axis=0,
                       keepdims=True)
    uniq_i = uniq_row.astype(jnp.int32)
    uniq257 = jnp.concatenate(
        [jnp.zeros((1, 1), jnp.int32), uniq_i], axis=1)
    mask257 = jnp.concatenate(
        [jnp.ones((1, 1), jnp.int32), (uniq_i != 0).astype(jnp.int32)], axis=1)

    uniq_ref[...] = uniq257
    maskout_ref[...] = mask257
    tail = uniq257[:, _K:_K + 1]
    uniq_ext = jnp.concatenate(
        [uniq257, jnp.broadcast_to(tail, (1, _CPH * _CHUNK - _K - 1))],
        axis=1)
    hh = lax.broadcasted_iota(jnp.int32, (_H, _CPH * _CHUNK), 0)
    idx_ref[...] = hh * _N + uniq_ext


def _sampling_call(cls_row, value3, maskf, g2):
    return pl.pallas_call(
        _sample_body,
        out_shape=[
            jax.ShapeDtypeStruct((1, _K + 1), jnp.int32),
            jax.ShapeDtypeStruct((1, _K + 1), jnp.int32),
            jax.ShapeDtypeStruct((_H, _CPH * _CHUNK), jnp.int32),
        ],
    )(cls_row, value3, maskf, g2)


def _g2_compute():
    gkey = jax.random.fold_in(jax.random.key(0), 123)
    u = jax.random.uniform(gkey, (1, _K, _N - 1), dtype=jnp.float32)
    gum = -jnp.log(-jnp.log(u + _EPS) + _EPS)
    return jnp.concatenate(
        [jnp.full((_K, 1), _NEG_BIG, jnp.float32),
         gum.reshape(_K, _N - 1)], axis=1)


_G2_CACHE = None


def _g2_const():
    global _G2_CACHE
    if _G2_CACHE is None:
        try:
            with jax.ensure_compile_time_eval():
                _G2_CACHE = np.asarray(jax.device_get(_g2_compute()))
        except Exception:
            return _g2_compute()
    return jnp.asarray(_G2_CACHE)


@functools.cache
def _jidx_const():
    rows = np.minimum(np.arange(_CPH * _CHUNK), _K).astype(np.int32)
    pat = rows.reshape(_CPH, _CHUNK)
    c = np.arange(_NW * _CPW)
    g = pat[c % _CPH] + (c // _CPH).clip(max=_H - 1)[:, None] * (_CPH * _CHUNK)
    return g.astype(np.int32).reshape(_NW, _CPW, _CHUNK)


@functools.cache
def _make_gather():
    @functools.partial(
        pl.kernel,
        mesh=plsc.VectorSubcoreMesh(core_axis_name="c", subcore_axis_name="s"),
        out_type=jax.ShapeDtypeStruct((_H * _CPH * _CHUNK, _N), jnp.float32),
        scratch_types=[
            pltpu.VMEM((_CPW * _CHUNK,), jnp.int32),
            pltpu.VMEM((_CPW, _CHUNK), jnp.int32),
            pltpu.VMEM((_CHUNK, _N), jnp.float32),
            pltpu.VMEM((_CHUNK, _N), jnp.float32),
            pltpu.VMEM((_CHUNK, _N), jnp.float32),
            pltpu.VMEM((_CHUNK, _N), jnp.float32),
            pltpu.SemaphoreType.DMA,
            pltpu.SemaphoreType.DMA,
            pltpu.SemaphoreType.DMA,
            pltpu.SemaphoreType.DMA,
            pltpu.SemaphoreType.DMA,
            pltpu.SemaphoreType.DMA,
            pltpu.SemaphoreType.DMA,
            pltpu.SemaphoreType.DMA,
        ],
    )
    def _gather_rows(table_hbm, idx_hbm, jidx_hbm, out_hbm, idx_v, jidx_v,
                     buf0, buf1, buf2, buf3, gsem0, gsem1, gsem2, gsem3,
                     ssem0, ssem1, ssem2, ssem3):
        wid = lax.axis_index("s") * _NC + lax.axis_index("c")
        base = wid * _CPW * _CHUNK
        pltpu.sync_copy(idx_hbm.at[pl.ds(base, _CPW * _CHUNK)], idx_v)
        pltpu.sync_copy(jidx_hbm.at[wid], jidx_v)
        bufs = (buf0, buf1, buf2, buf3)
        gsems = (gsem0, gsem1, gsem2, gsem3)
        ssems = (ssem0, ssem1, ssem2, ssem3)

        def chunk_of(t):
            return wid * _CPW + t

        def gather_desc(t):
            return pltpu.make_async_copy(
                table_hbm.at[idx_v.at[pl.ds(t * _CHUNK, _CHUNK)]],
                bufs[t % 4], gsems[t % 4])

        def scatter_desc(t):
            return pltpu.make_async_copy(
                bufs[t % 4], out_hbm.at[jidx_v.at[t]], ssems[t % 4])

        def start_gather(t):
            @pl.when(chunk_of(t) < _NCH)
            def _():
                gather_desc(t).start()

        def wait_gather(t):
            @pl.when(chunk_of(t) < _NCH)
            def _():
                gather_desc(t).wait()

        def start_scatter(t):
            @pl.when(chunk_of(t) < _NCH)
            def _():
                scatter_desc(t).start()

        def wait_scatter(t):
            @pl.when(chunk_of(t) < _NCH)
            def _():
                scatter_desc(t).wait()

        start_gather(0)
        start_gather(1)
        for t in range(_CPW):
            if t >= 2:
                wait_scatter(t - 2)
            if t + 2 < _CPW:
                start_gather(t + 2)
            wait_gather(t)
            start_scatter(t)
        wait_scatter(_CPW - 2)
        wait_scatter(_CPW - 1)

    return _gather_rows


def kernel(attn, value, mask):
    b, h, n, _ = attn.shape
    k = _K
    g2 = _g2_const()

    cls_row = attn[0, :, 0, :]
    value3 = value[0]
    maskf = mask.astype(jnp.float32)

    uniq257, mask257, idx264 = _sampling_call(cls_row, value3, maskf, g2)

    idx_flat = jnp.pad(idx264.reshape(_H * _CPH * _CHUNK),
                       (0, _IDXN - _H * _CPH * _CHUNK))
    jidx = jnp.asarray(_jidx_const())
    table = attn.reshape(h * n, n)

    out = _make_gather()(table, idx_flat, jidx)
    new_attn = out.reshape(_H, _CPH * _CHUNK, n)[:, :k + 1, :].reshape(
        1, h, k + 1, n)
    return new_attn, mask257.astype(bool), uniq257


# speedup vs baseline: 1.1548x; 1.0144x over previous
import functools

import jax
import jax.numpy as jnp
import numpy as np
from jax import lax
from jax.experimental import pallas as pl
from jax.experimental.pallas import tpu as pltpu
from jax.experimental.pallas import tpu_sc as plsc

_K = 256
_N = 2048
_H = 12
_DH = 64
_EPS = 1e-06
_MASK_VAL = float(np.finfo(np.float32).max) / 2
_NEG_BIG = -1e38

_NC = 2
_NS = 16
_NW = _NC * _NS
_CHUNK = 8
_CPH = 33
_NCH = _H * _CPH
_CPW = 13
_IDXN = _NW * _CPW * _CHUNK


def _sample_body(cls_ref, val_ref, maskf_ref, g2_ref, uniq_ref, maskout_ref,
                 idx_ref):
    cls = cls_ref[...]
    ones_dh = jnp.ones((1, _DH), jnp.float32)
    s = jnp.zeros((1, _N), jnp.float32)
    for hh0 in range(_H):
        vh = val_ref[hh0]
        n2 = lax.dot_general(ones_dh, vh * vh,
                             (((1,), (1,)), ((), ())),
                             preferred_element_type=jnp.float32)
        s = s + cls[hh0:hh0 + 1, :] * jnp.sqrt(n2)

    it = lax.broadcasted_iota(jnp.int32, (1, _N), 1)
    validf = (it >= 1).astype(jnp.float32)
    total = jnp.sum(s * validf)
    logits = jnp.log(s / (total + _EPS) + _EPS)
    logits = jnp.where(maskf_ref[...] > 0.5, logits, -_MASK_VAL)

    score = logits + g2_ref[...]
    m = jnp.max(score, axis=1, keepdims=True)
    ti = lax.broadcasted_iota(jnp.int32, (_K, _N), 1)
    sampled = jnp.min(jnp.where(score == m, ti, _N * 2), axis=1,
                      keepdims=True)

    presentf = jnp.max((sampled == ti).astype(jnp.float32), axis=0,
                       keepdims=True)
    n_uniq = jnp.sum(presentf)

    it0 = lax.broadcasted_iota(jnp.int32, (_N, _N), 0)
    it1 = lax.broadcasted_iota(jnp.int32, (_N, _N), 1)
    tri = (it1 <= it0).astype(jnp.float32)
    ranks_col = lax.dot_general(tri, presentf, (((1,), (1,)), ((), ())),
                                preferred_element_type=jnp.float32)
    ranks_excl = jnp.concatenate(
        [jnp.zeros((1, 1), jnp.float32), ranks_col[: _N - 1, :]], axis=0)
    present_col = ranks_col - ranks_excl

    slot_col = ranks_col + (float(_K) - 1.0 - n_uniq)
    jj = lax.broadcasted_iota(jnp.int32, (_N, _K), 1).astype(jnp.float32)
    tvals = lax.broadcasted_iota(jnp.int32, (_N, _K), 0).astype(jnp.float32)
    hit = (slot_col == jj) & (present_col > 0.5)
    uniq_row = jnp.sum(jnp.where(hit, tvals, 0.0), axis=0,
                       keepdims=True)
    uniq_i = uniq_row.astype(jnp.int32)
    uniq257 = jnp.concatenate(
        [jnp.zeros((1, 1), jnp.int32), uniq_i], axis=1)
    mask257 = jnp.concatenate(
        [jnp.ones((1, 1), jnp.int32), (uniq_i != 0).astype(jnp.int32)], axis=1)

    uniq_ref[...] = uniq257
    maskout_ref[...] = mask257
    tail = uniq257[:, _K:_K + 1]
    uniq_ext = jnp.concatenate(
        [uniq257, jnp.broadcast_to(tail, (1, _CPH * _CHUNK - _K - 1))],
        axis=1)
    hh = lax.broadcasted_iota(jnp.int32, (_H, _CPH * _CHUNK), 0)
    idx_ref[...] = hh * _N + uniq_ext


def _sampling_call(cls_row, value3, maskf, g2):
    return pl.pallas_call(
        _sample_body,
        out_shape=[
            jax.ShapeDtypeStruct((1, _K + 1), jnp.int32),
            jax.ShapeDtypeStruct((1, _K + 1), jnp.int32),
            jax.ShapeDtypeStruct((_H, _CPH * _CHUNK), jnp.int32),
        ],
    )(cls_row, value3, maskf, g2)


def _g2_compute():
    gkey = jax.random.fold_in(jax.random.key(0), 123)
    u = jax.random.uniform(gkey, (1, _K, _N - 1), dtype=jnp.float32)
    gum = -jnp.log(-jnp.log(u + _EPS) + _EPS)
    return jnp.concatenate(
        [jnp.full((_K, 1), _NEG_BIG, jnp.float32),
         gum.reshape(_K, _N - 1)], axis=1)


_G2_CACHE = None


def _g2_const():
    global _G2_CACHE
    if _G2_CACHE is None:
        try:
            with jax.ensure_compile_time_eval():
                _G2_CACHE = np.asarray(jax.device_get(_g2_compute()))
        except Exception:
            return _g2_compute()
    return jnp.asarray(_G2_CACHE)


@functools.cache
def _jidx_const():
    rows = np.minimum(np.arange(_CPH * _CHUNK), _K).astype(np.int32)
    pat = rows.reshape(_CPH, _CHUNK)
    c = np.arange(_NW * _CPW)
    g = pat[c % _CPH] + (c // _CPH).clip(max=_H - 1)[:, None] * (_CPH * _CHUNK)
    return g.astype(np.int32).reshape(_NW, _CPW, _CHUNK)


@functools.cache
def _make_gather():
    @functools.partial(
        pl.kernel,
        mesh=plsc.VectorSubcoreMesh(core_axis_name="c", subcore_axis_name="s"),
        out_type=jax.ShapeDtypeStruct((_H * _CPH * _CHUNK, _N), jnp.float32),
        scratch_types=[
            pltpu.VMEM((_CPW * _CHUNK,), jnp.int32),
            pltpu.VMEM((_CPW, _CHUNK), jnp.int32),
            pltpu.VMEM((_CHUNK, _N), jnp.float32),
            pltpu.VMEM((_CHUNK, _N), jnp.float32),
            pltpu.VMEM((_CHUNK, _N), jnp.float32),
            pltpu.VMEM((_CHUNK, _N), jnp.float32),
            pltpu.SemaphoreType.DMA,
            pltpu.SemaphoreType.DMA,
            pltpu.SemaphoreType.DMA,
            pltpu.SemaphoreType.DMA,
            pltpu.SemaphoreType.DMA,
            pltpu.SemaphoreType.DMA,
            pltpu.SemaphoreType.DMA,
            pltpu.SemaphoreType.DMA,
        ],
    )
    def _gather_rows(table_hbm, idx_hbm, jidx_hbm, out_hbm, idx_v, jidx_v,
                     buf0, buf1, buf2, buf3, gsem0, gsem1, gsem2, gsem3,
                     ssem0, ssem1, ssem2, ssem3):
        wid = lax.axis_index("s") * _NC + lax.axis_index("c")
        base = wid * _CPW * _CHUNK
        pltpu.sync_copy(idx_hbm.at[pl.ds(base, _CPW * _CHUNK)], idx_v)
        pltpu.sync_copy(jidx_hbm.at[wid], jidx_v)
        bufs = (buf0, buf1, buf2, buf3)
        gsems = (gsem0, gsem1, gsem2, gsem3)
        ssems = (ssem0, ssem1, ssem2, ssem3)

        def chunk_of(t):
            return wid * _CPW + t

        def gather_desc(t):
            return pltpu.make_async_copy(
                table_hbm.at[idx_v.at[pl.ds(t * _CHUNK, _CHUNK)]],
                bufs[t % 4], gsems[t % 4])

        def scatter_desc(t):
            return pltpu.make_async_copy(
                bufs[t % 4], out_hbm.at[jidx_v.at[t]], ssems[t % 4])

        def start_gather(t):
            @pl.when(chunk_of(t) < _NCH)
            def _():
                gather_desc(t).start()

        def wait_gather(t):
            @pl.when(chunk_of(t) < _NCH)
            def _():
                gather_desc(t).wait()

        def start_scatter(t):
            @pl.when(chunk_of(t) < _NCH)
            def _():
                scatter_desc(t).start()

        def wait_scatter(t):
            @pl.when(chunk_of(t) < _NCH)
            def _():
                scatter_desc(t).wait()

        start_gather(0)
        start_gather(1)
        for t in range(_CPW):
            if t >= 2:
                wait_scatter(t - 2)
            if t + 2 < _CPW:
                start_gather(t + 2)
            wait_gather(t)
            start_scatter(t)
        wait_scatter(_CPW - 2)
        wait_scatter(_CPW - 1)

    return _gather_rows


def kernel(attn, value, mask):
    b, h, n, _ = attn.shape
    k = _K
    g2 = _g2_const()

    cls_row = attn[0, :, 0, :]
    value3 = value[0]
    maskf = mask.astype(jnp.float32)

    uniq257, mask257, idx264 = _sampling_call(cls_row, value3, maskf, g2)

    idx_flat = jnp.pad(idx264.reshape(_H * _CPH * _CHUNK),
                       (0, _IDXN - _H * _CPH * _CHUNK))
    jidx = jnp.asarray(_jidx_const())
    table = attn.reshape(h * n, n)

    out = _make_gather()(table, idx_flat, jidx)
    new_attn = out.reshape(_H, _CPH * _CHUNK, n)[:, :k + 1, :].reshape(
        1, h, k + 1, n)
    return new_attn, mask257.astype(bool), uniq257


# speedup vs baseline: 1.1859x; 1.0269x over previous
import functools

import jax
import jax.numpy as jnp
import numpy as np
from jax import lax
from jax.experimental import pallas as pl
from jax.experimental.pallas import tpu as pltpu
from jax.experimental.pallas import tpu_sc as plsc

_K = 256
_N = 2048
_H = 12
_DH = 64
_EPS = 1e-06
_MASK_VAL = float(np.finfo(np.float32).max) / 2
_NEG_BIG = -1e38

_NC = 2
_NS = 16
_NW = _NC * _NS
_CHUNK = 8
_CPH = 33
_NCH = _H * _CPH
_CPW = 13
_IDXN = _NW * _CPW * _CHUNK


def _sample_body(cls_ref, val_ref, maskf_ref, g2_ref, uniq_ref, maskout_ref,
                 idx_ref):
    cls = cls_ref[...]
    ones_dh = jnp.ones((1, _DH), jnp.float32)
    s = jnp.zeros((1, _N), jnp.float32)
    for hh0 in range(_H):
        vh = val_ref[hh0]
        n2 = lax.dot_general(ones_dh, vh * vh,
                             (((1,), (1,)), ((), ())),
                             preferred_element_type=jnp.float32)
        s = s + cls[hh0:hh0 + 1, :] * jnp.sqrt(n2)

    it = lax.broadcasted_iota(jnp.int32, (1, _N), 1)
    validf = (it >= 1).astype(jnp.float32)
    total = jnp.sum(s * validf)
    logits = jnp.log(s / (total + _EPS) + _EPS)
    logits = jnp.where(maskf_ref[...] > 0.5, logits, -_MASK_VAL)

    score = logits + g2_ref[...]
    m = jnp.max(score, axis=1, keepdims=True)
    ti = lax.broadcasted_iota(jnp.int32, (_K, _N), 1)
    sampled = jnp.min(jnp.where(score == m, ti, _N * 2), axis=1,
                      keepdims=True)

    presentf = jnp.max((sampled == ti).astype(jnp.float32), axis=0,
                       keepdims=True)
    n_uniq = jnp.sum(presentf)

    it0 = lax.broadcasted_iota(jnp.int32, (_N, _N), 0)
    it1 = lax.broadcasted_iota(jnp.int32, (_N, _N), 1)
    tri = (it1 <= it0).astype(jnp.float32)
    ranks_col = lax.dot_general(tri, presentf, (((1,), (1,)), ((), ())),
                                preferred_element_type=jnp.float32)
    ranks_excl = jnp.concatenate(
        [jnp.zeros((1, 1), jnp.float32), ranks_col[: _N - 1, :]], axis=0)
    present_col = ranks_col - ranks_excl

    slot_col = ranks_col + (float(_K) - 1.0 - n_uniq)
    jj = lax.broadcasted_iota(jnp.int32, (_N, _K), 1).astype(jnp.float32)
    tvals = lax.broadcasted_iota(jnp.int32, (_N, _K), 0).astype(jnp.float32)
    hit = (slot_col == jj) & (present_col > 0.5)
    uniq_row = jnp.sum(jnp.where(hit, tvals, 0.0), axis=0,
                       keepdims=True)
    uniq_i = uniq_row.astype(jnp.int32)
    uniq257 = jnp.concatenate(
        [jnp.zeros((1, 1), jnp.int32), uniq_i], axis=1)
    mask257 = jnp.concatenate(
        [jnp.ones((1, 1), jnp.int32), (uniq_i != 0).astype(jnp.int32)], axis=1)

    uniq_ref[...] = uniq257
    maskout_ref[...] = mask257
    tail = uniq257[:, _K:_K + 1]
    uniq_ext = jnp.concatenate(
        [uniq257, jnp.broadcast_to(tail, (1, _CPH * _CHUNK - _K - 1))],
        axis=1)
    hh = lax.broadcasted_iota(jnp.int32, (_H, _CPH * _CHUNK), 0)
    idx_ref[...] = hh * _N + uniq_ext


def _sampling_call(cls_row, value3, maskf, g2):
    return pl.pallas_call(
        _sample_body,
        out_shape=[
            jax.ShapeDtypeStruct((1, _K + 1), jnp.int32),
            jax.ShapeDtypeStruct((1, _K + 1), jnp.int32),
            jax.ShapeDtypeStruct((_H, _CPH * _CHUNK), jnp.int32),
        ],
    )(cls_row, value3, maskf, g2)


def _g2_compute():
    gkey = jax.random.fold_in(jax.random.key(0), 123)
    u = jax.random.uniform(gkey, (1, _K, _N - 1), dtype=jnp.float32)
    gum = -jnp.log(-jnp.log(u + _EPS) + _EPS)
    return jnp.concatenate(
        [jnp.full((_K, 1), _NEG_BIG, jnp.float32),
         gum.reshape(_K, _N - 1)], axis=1)


_G2_CACHE = None


def _g2_const():
    global _G2_CACHE
    if _G2_CACHE is None:
        try:
            with jax.ensure_compile_time_eval():
                _G2_CACHE = np.asarray(jax.device_get(_g2_compute()))
        except Exception:
            return _g2_compute()
    return jnp.asarray(_G2_CACHE)


@functools.cache
def _make_gather():
    @functools.partial(
        pl.kernel,
        mesh=plsc.VectorSubcoreMesh(core_axis_name="c", subcore_axis_name="s"),
        out_type=jax.ShapeDtypeStruct((_H * _CPH * _CHUNK, _N), jnp.float32),
        scratch_types=[
            pltpu.VMEM((_CPW * _CHUNK,), jnp.int32),
            pltpu.VMEM((_CHUNK, _N), jnp.float32),
            pltpu.VMEM((_CHUNK, _N), jnp.float32),
            pltpu.VMEM((_CHUNK, _N), jnp.float32),
            pltpu.VMEM((_CHUNK, _N), jnp.float32),
            pltpu.SemaphoreType.DMA,
            pltpu.SemaphoreType.DMA,
            pltpu.SemaphoreType.DMA,
            pltpu.SemaphoreType.DMA,
            pltpu.SemaphoreType.DMA,
            pltpu.SemaphoreType.DMA,
            pltpu.SemaphoreType.DMA,
            pltpu.SemaphoreType.DMA,
        ],
    )
    def _gather_rows(table_hbm, idx_hbm, out_hbm, idx_v,
                     buf0, buf1, buf2, buf3, gsem0, gsem1, gsem2, gsem3,
                     ssem0, ssem1, ssem2, ssem3):
        wid = lax.axis_index("s") * _NC + lax.axis_index("c")
        base = wid * _CPW * _CHUNK
        pltpu.sync_copy(idx_hbm.at[pl.ds(base, _CPW * _CHUNK)], idx_v)
        bufs = (buf0, buf1, buf2, buf3)
        gsems = (gsem0, gsem1, gsem2, gsem3)
        ssems = (ssem0, ssem1, ssem2, ssem3)

        def chunk_of(t):
            return wid * _CPW + t

        def gather_desc(t):
            return pltpu.make_async_copy(
                table_hbm.at[idx_v.at[pl.ds(t * _CHUNK, _CHUNK)]],
                bufs[t % 4], gsems[t % 4])

        def scatter_desc(t):
            return pltpu.make_async_copy(
                bufs[t % 4],
                out_hbm.at[pl.ds(chunk_of(t) * _CHUNK, _CHUNK)],
                ssems[t % 4])

        def start_gather(t):
            @pl.when(chunk_of(t) < _NCH)
            def _():
                gather_desc(t).start()

        def wait_gather(t):
            @pl.when(chunk_of(t) < _NCH)
            def _():
                gather_desc(t).wait()

        def start_scatter(t):
            @pl.when(chunk_of(t) < _NCH)
            def _():
                scatter_desc(t).start()

        def wait_scatter(t):
            @pl.when(chunk_of(t) < _NCH)
            def _():
                scatter_desc(t).wait()

        start_gather(0)
        start_gather(1)
        for t in range(_CPW):
            if t >= 2:
                wait_scatter(t - 2)
            if t + 2 < _CPW:
                start_gather(t + 2)
            wait_gather(t)
            start_scatter(t)
        wait_scatter(_CPW - 2)
        wait_scatter(_CPW - 1)

    return _gather_rows


def kernel(attn, value, mask):
    b, h, n, _ = attn.shape
    k = _K
    g2 = _g2_const()

    cls_row = attn[0, :, 0, :]
    value3 = value[0]
    maskf = mask.astype(jnp.float32)

    uniq257, mask257, idx264 = _sampling_call(cls_row, value3, maskf, g2)

    idx_flat = jnp.pad(idx264.reshape(_H * _CPH * _CHUNK),
                       (0, _IDXN - _H * _CPH * _CHUNK))
    table = attn.reshape(h * n, n)

    out = _make_gather()(table, idx_flat)
    new_attn = out.reshape(_H, _CPH * _CHUNK, n)[:, :k + 1, :].reshape(
        1, h, k + 1, n)
    return new_attn, mask257.astype(bool), uniq257
